# baseline (speedup 1.0000x reference)
import numpy as np
import ml_dtypes
import jax
import jax.numpy as jnp
from jax.experimental import pallas as pl
from jax.experimental.pallas import tpu as pltpu

_BB = 256


def _sel(n_out, n_in, k, off):
    r = np.zeros((n_out, n_in, k), np.float32)
    p = np.arange(n_out)[:, None]
    t = np.arange(k)[None, :]
    r[p, 2 * p + off + t, t] = 1.0
    return r


_R1 = [_sel(12, 28, 5, d) for d in (0, 1)]
_R2 = [_sel(4, 12, 5, d) for d in (0, 1)]

_P1 = np.stack([
    np.einsum('prh,qsw->rspqhw', _R1[dh], _R1[dw]).reshape(784, 144, 25)
    for dh in (0, 1) for dw in (0, 1)
]).reshape(4 * 784 * 144, 25).astype(ml_dtypes.bfloat16)

_P2 = np.stack([
    np.einsum('aph,bqw->pqabhw', _R2[dh], _R2[dw]).reshape(144, 16, 25)
    for dh in (0, 1) for dw in (0, 1)
]).reshape(4 * 144 * 16, 25).astype(ml_dtypes.bfloat16)


def _net_kernel(x_ref, g_ref, h_ref, b1_ref, b2_ref,
                f1_ref, fb1_ref, f2_ref, fb2_ref, o_ref):
    x = x_ref[...].astype(jnp.bfloat16)

    z = jnp.dot(x, g_ref[0], preferred_element_type=jnp.float32)
    z = jnp.maximum(z, jnp.dot(x, g_ref[1], preferred_element_type=jnp.float32))
    z = jnp.maximum(z, jnp.dot(x, g_ref[2], preferred_element_type=jnp.float32))
    z = jnp.maximum(z, jnp.dot(x, g_ref[3], preferred_element_type=jnp.float32))
    a1 = jnp.maximum(z + b1_ref[...], 0.0).astype(jnp.bfloat16)

    z = jnp.dot(a1, h_ref[0], preferred_element_type=jnp.float32)
    z = jnp.maximum(z, jnp.dot(a1, h_ref[1], preferred_element_type=jnp.float32))
    z = jnp.maximum(z, jnp.dot(a1, h_ref[2], preferred_element_type=jnp.float32))
    z = jnp.maximum(z, jnp.dot(a1, h_ref[3], preferred_element_type=jnp.float32))
    a2 = jnp.maximum(z + b2_ref[...], 0.0).astype(jnp.bfloat16)

    h = jnp.dot(a2, f1_ref[...], preferred_element_type=jnp.float32) + fb1_ref[...]
    h = jnp.maximum(h, 0.0).astype(jnp.bfloat16)
    logits = jnp.dot(h, f2_ref[...], preferred_element_type=jnp.float32) + fb2_ref[...]
    m = jnp.max(logits, axis=-1, keepdims=True)
    lse = jnp.log(jnp.sum(jnp.exp(logits - m), axis=-1, keepdims=True)) + m
    o_ref[...] = logits - lse


def kernel(c1_w, c1_b, c2_w, c2_b, fc1_w, fc1_b, fc2_w, fc2_b, x):
    B = x.shape[0]
    xr = x.reshape(B, 28 * 28)

    w1 = c1_w[:, :10].astype(jnp.bfloat16)
    g_all = (jnp.dot(jnp.asarray(_P1), w1, preferred_element_type=jnp.float32)
             .astype(jnp.bfloat16).reshape(4, 784, 1440))
    w2 = c2_w[:, :20].reshape(25, 200).astype(jnp.bfloat16)
    h_all = (jnp.dot(jnp.asarray(_P2), w2, preferred_element_type=jnp.float32)
             .astype(jnp.bfloat16).reshape(4, 144, 16, 10, 20)
             .transpose(0, 1, 3, 2, 4).reshape(4, 1440, 320))

    b1l = jnp.tile(c1_b[0, :10], 144).reshape(1, 1440)
    b2l = jnp.tile(c2_b[0, :20], 16).reshape(1, 320)
    f1 = fc1_w.astype(jnp.bfloat16)
    f2 = fc2_w.astype(jnp.bfloat16)

    const = lambda shape: pl.BlockSpec(shape, lambda i: tuple(0 for _ in shape))
    out = pl.pallas_call(
        _net_kernel,
        out_shape=jax.ShapeDtypeStruct((B, 128), jnp.float32),
        grid=(B // _BB,),
        in_specs=[pl.BlockSpec((_BB, 784), lambda i: (i, 0)),
                  const((4, 784, 1440)), const((4, 1440, 320)),
                  const((1, 1440)), const((1, 320)),
                  const((320, 128)), const((1, 128)),
                  const((128, 128)), const((1, 128))],
        out_specs=pl.BlockSpec((_BB, 128), lambda i: (i, 0)),
        compiler_params=pltpu.CompilerParams(dimension_semantics=("parallel",)),
    )(xr, g_all, h_all, b1l, b2l, f1, fc1_b, f2, fc2_b)
    return out[:, :10]


# speedup vs baseline: 1.0723x; 1.0723x over previous
import numpy as np
import ml_dtypes
import jax
import jax.numpy as jnp
from jax.experimental import pallas as pl
from jax.experimental.pallas import tpu as pltpu

_BB = 256


def _sel(n_out, n_in, k, off):
    r = np.zeros((n_out, n_in, k), np.float32)
    p = np.arange(n_out)[:, None]
    t = np.arange(k)[None, :]
    r[p, 2 * p + off + t, t] = 1.0
    return r


_R1 = [_sel(12, 28, 5, d) for d in (0, 1)]
_R2 = [_sel(4, 12, 5, d) for d in (0, 1)]

_R1S = np.stack(_R1)
_R2S = np.stack(_R2)


def _net_kernel(x_ref, g_ref, h_ref, b1_ref, b2_ref,
                f1_ref, fb1_ref, f2_ref, fb2_ref, o_ref):
    x = x_ref[...].astype(jnp.bfloat16)

    z = jnp.dot(x, g_ref[0], preferred_element_type=jnp.float32)
    z = jnp.maximum(z, jnp.dot(x, g_ref[1], preferred_element_type=jnp.float32))
    z = jnp.maximum(z, jnp.dot(x, g_ref[2], preferred_element_type=jnp.float32))
    z = jnp.maximum(z, jnp.dot(x, g_ref[3], preferred_element_type=jnp.float32))
    a1 = jnp.maximum(z + b1_ref[...], 0.0).astype(jnp.bfloat16)

    z = jnp.dot(a1, h_ref[0], preferred_element_type=jnp.float32)
    z = jnp.maximum(z, jnp.dot(a1, h_ref[1], preferred_element_type=jnp.float32))
    z = jnp.maximum(z, jnp.dot(a1, h_ref[2], preferred_element_type=jnp.float32))
    z = jnp.maximum(z, jnp.dot(a1, h_ref[3], preferred_element_type=jnp.float32))
    a2 = jnp.maximum(z + b2_ref[...], 0.0).astype(jnp.bfloat16)

    h = jnp.dot(a2, f1_ref[...], preferred_element_type=jnp.float32) + fb1_ref[...]
    h = jnp.maximum(h, 0.0).astype(jnp.bfloat16)
    logits = jnp.dot(h, f2_ref[...], preferred_element_type=jnp.float32) + fb2_ref[...]
    m = jnp.max(logits, axis=-1, keepdims=True)
    lse = jnp.log(jnp.sum(jnp.exp(logits - m), axis=-1, keepdims=True)) + m
    o_ref[...] = logits - lse


def kernel(c1_w, c1_b, c2_w, c2_b, fc1_w, fc1_b, fc2_w, fc2_b, x):
    B = x.shape[0]
    xr = x.reshape(B, 28 * 28)

    w1 = c1_w[:, :10].reshape(5, 5, 10)
    a1op = jnp.einsum('eqsw,hwc->eqshc', _R1S, w1)
    g_big = jnp.einsum('dprh,eqshc->dpreqsc', _R1S, a1op)
    g_all = (g_big.transpose(0, 3, 2, 5, 1, 4, 6)
             .reshape(4, 784, 1440).astype(jnp.bfloat16))
    w2 = c2_w[:, :20].reshape(5, 5, 10, 20)
    a2op = jnp.einsum('ebqw,hwic->ebqhic', _R2S, w2)
    h_big = jnp.einsum('daph,ebqhic->dapebqic', _R2S, a2op)
    h_all = (h_big.transpose(0, 3, 2, 5, 6, 1, 4, 7)
             .reshape(4, 1440, 320).astype(jnp.bfloat16))

    b1l = jnp.tile(c1_b[0, :10], 144).reshape(1, 1440)
    b2l = jnp.tile(c2_b[0, :20], 16).reshape(1, 320)
    f1 = fc1_w.astype(jnp.bfloat16)
    f2 = fc2_w.astype(jnp.bfloat16)

    const = lambda shape: pl.BlockSpec(shape, lambda i: tuple(0 for _ in shape))
    out = pl.pallas_call(
        _net_kernel,
        out_shape=jax.ShapeDtypeStruct((B, 128), jnp.float32),
        grid=(B // _BB,),
        in_specs=[pl.BlockSpec((_BB, 784), lambda i: (i, 0)),
                  const((4, 784, 1440)), const((4, 1440, 320)),
                  const((1, 1440)), const((1, 320)),
                  const((320, 128)), const((1, 128)),
                  const((128, 128)), const((1, 128))],
        out_specs=pl.BlockSpec((_BB, 128), lambda i: (i, 0)),
        compiler_params=pltpu.CompilerParams(dimension_semantics=("parallel",)),
    )(xr, g_all, h_all, b1l, b2l, f1, fc1_b, f2, fc2_b)
    return out[:, :10]


# speedup vs baseline: 1.2882x; 1.2013x over previous
import numpy as np
import ml_dtypes
import jax
import jax.numpy as jnp
from jax.experimental import pallas as pl
from jax.experimental.pallas import tpu as pltpu

_BB = 256


def _sel(n_out, n_in, k, off):
    r = np.zeros((n_out, n_in, k), np.float32)
    p = np.arange(n_out)[:, None]
    t = np.arange(k)[None, :]
    r[p, 2 * p + off + t, t] = 1.0
    return r


_R1 = [_sel(12, 28, 5, d) for d in (0, 1)]
_R2 = [_sel(4, 12, 5, d) for d in (0, 1)]

_R1S = np.stack(_R1)
_R2S = np.stack(_R2)


def _net_kernel(x_ref, g_ref, h_ref, b1_ref, b2_ref,
                f1_ref, fb1_ref, f2_ref, fb2_ref, o_ref):
    x = x_ref[...].reshape(x_ref.shape[0], 784).astype(jnp.bfloat16)

    z = jnp.dot(x, g_ref[0], preferred_element_type=jnp.float32)
    z = jnp.maximum(z, jnp.dot(x, g_ref[1], preferred_element_type=jnp.float32))
    z = jnp.maximum(z, jnp.dot(x, g_ref[2], preferred_element_type=jnp.float32))
    z = jnp.maximum(z, jnp.dot(x, g_ref[3], preferred_element_type=jnp.float32))
    a1 = jnp.maximum(z + b1_ref[...], 0.0).astype(jnp.bfloat16)

    z = jnp.dot(a1, h_ref[0], preferred_element_type=jnp.float32)
    z = jnp.maximum(z, jnp.dot(a1, h_ref[1], preferred_element_type=jnp.float32))
    z = jnp.maximum(z, jnp.dot(a1, h_ref[2], preferred_element_type=jnp.float32))
    z = jnp.maximum(z, jnp.dot(a1, h_ref[3], preferred_element_type=jnp.float32))
    a2 = jnp.maximum(z + b2_ref[...], 0.0).astype(jnp.bfloat16)

    h = jnp.dot(a2, f1_ref[...], preferred_element_type=jnp.float32) + fb1_ref[...]
    h = jnp.maximum(h, 0.0).astype(jnp.bfloat16)
    logits = jnp.dot(h, f2_ref[...], preferred_element_type=jnp.float32) + fb2_ref[...]
    m = jnp.max(logits, axis=-1, keepdims=True)
    lse = jnp.log(jnp.sum(jnp.exp(logits - m), axis=-1, keepdims=True)) + m
    o_ref[...] = logits - lse


def kernel(c1_w, c1_b, c2_w, c2_b, fc1_w, fc1_b, fc2_w, fc2_b, x):
    B = x.shape[0]
    xr = x.reshape(B, 28, 28)

    w1 = c1_w[:, :10].reshape(5, 5, 10)
    a1op = jnp.einsum('eqsw,hwc->eqshc', _R1S, w1)
    g_big = jnp.einsum('dprh,eqshc->dpreqsc', _R1S, a1op)
    g_all = (g_big.transpose(0, 3, 2, 5, 1, 4, 6)
             .reshape(4, 784, 1440).astype(jnp.bfloat16))
    w2 = c2_w[:, :20].reshape(5, 5, 10, 20)
    a2op = jnp.einsum('ebqw,hwic->ebqhic', _R2S, w2)
    h_big = jnp.einsum('daph,ebqhic->dapebqic', _R2S, a2op)
    h_all = (h_big.transpose(0, 3, 2, 5, 6, 1, 4, 7)
             .reshape(4, 1440, 320).astype(jnp.bfloat16))

    b1l = jnp.tile(c1_b[0, :10], 144).reshape(1, 1440)
    b2l = jnp.tile(c2_b[0, :20], 16).reshape(1, 320)
    f1 = fc1_w.astype(jnp.bfloat16)
    f2 = fc2_w.astype(jnp.bfloat16)

    const = lambda shape: pl.BlockSpec(shape, lambda i: tuple(0 for _ in shape))
    out = pl.pallas_call(
        _net_kernel,
        out_shape=jax.ShapeDtypeStruct((B, 128), jnp.float32),
        grid=(B // _BB,),
        in_specs=[pl.BlockSpec((_BB, 28, 28), lambda i: (i, 0, 0)),
                  const((4, 784, 1440)), const((4, 1440, 320)),
                  const((1, 1440)), const((1, 320)),
                  const((320, 128)), const((1, 128)),
                  const((128, 128)), const((1, 128))],
        out_specs=pl.BlockSpec((_BB, 128), lambda i: (i, 0)),
        compiler_params=pltpu.CompilerParams(dimension_semantics=("parallel",)),
    )(xr, g_all, h_all, b1l, b2l, f1, fc1_b, f2, fc2_b)
    return out[:, :10]


# speedup vs baseline: 1.3276x; 1.0306x over previous
import numpy as np
import ml_dtypes
import jax
import jax.numpy as jnp
from jax.experimental import pallas as pl
from jax.experimental.pallas import tpu as pltpu

_BB = 256


def _sel(n_out, n_in, k, off):
    r = np.zeros((n_out, n_in, k), np.float32)
    p = np.arange(n_out)[:, None]
    t = np.arange(k)[None, :]
    r[p, 2 * p + off + t, t] = 1.0
    return r


_R1 = [_sel(12, 28, 5, d) for d in (0, 1)]
_R2 = [_sel(4, 12, 5, d) for d in (0, 1)]

_R1S = np.stack(_R1)
_R2S = np.stack(_R2)


def _net_kernel(x_ref, g_ref, h_ref, b1_ref, b2_ref,
                f1_ref, fb1_ref, f2_ref, fb2_ref, o_ref):
    x = x_ref[...].astype(jnp.bfloat16).reshape(x_ref.shape[0], 784)

    z = jnp.dot(x, g_ref[0], preferred_element_type=jnp.float32).astype(jnp.bfloat16)
    z = jnp.maximum(z, jnp.dot(x, g_ref[1], preferred_element_type=jnp.float32).astype(jnp.bfloat16))
    z = jnp.maximum(z, jnp.dot(x, g_ref[2], preferred_element_type=jnp.float32).astype(jnp.bfloat16))
    z = jnp.maximum(z, jnp.dot(x, g_ref[3], preferred_element_type=jnp.float32).astype(jnp.bfloat16))
    a1 = jnp.maximum(z + b1_ref[...], 0.0)

    z = jnp.dot(a1, h_ref[0], preferred_element_type=jnp.float32).astype(jnp.bfloat16)
    z = jnp.maximum(z, jnp.dot(a1, h_ref[1], preferred_element_type=jnp.float32).astype(jnp.bfloat16))
    z = jnp.maximum(z, jnp.dot(a1, h_ref[2], preferred_element_type=jnp.float32).astype(jnp.bfloat16))
    z = jnp.maximum(z, jnp.dot(a1, h_ref[3], preferred_element_type=jnp.float32).astype(jnp.bfloat16))
    a2 = jnp.maximum(z + b2_ref[...], 0.0)

    h = jnp.dot(a2, f1_ref[...], preferred_element_type=jnp.float32) + fb1_ref[...]
    h = jnp.maximum(h, 0.0).astype(jnp.bfloat16)
    logits = jnp.dot(h, f2_ref[...], preferred_element_type=jnp.float32) + fb2_ref[...]
    m = jnp.max(logits, axis=-1, keepdims=True)
    lse = jnp.log(jnp.sum(jnp.exp(logits - m), axis=-1, keepdims=True)) + m
    o_ref[...] = logits - lse


def kernel(c1_w, c1_b, c2_w, c2_b, fc1_w, fc1_b, fc2_w, fc2_b, x):
    B = x.shape[0]
    xr = x.reshape(B, 28, 28)

    w1 = c1_w[:, :10].reshape(5, 5, 10)
    a1op = jnp.einsum('eqsw,hwc->eqshc', _R1S, w1)
    g_big = jnp.einsum('dprh,eqshc->dpreqsc', _R1S, a1op)
    g_all = (g_big.transpose(0, 3, 2, 5, 1, 4, 6)
             .reshape(4, 784, 1440).astype(jnp.bfloat16))
    w2 = c2_w[:, :20].reshape(5, 5, 10, 20)
    a2op = jnp.einsum('ebqw,hwic->ebqhic', _R2S, w2)
    h_big = jnp.einsum('daph,ebqhic->dapebqic', _R2S, a2op)
    h_all = (h_big.transpose(0, 3, 2, 5, 6, 1, 4, 7)
             .reshape(4, 1440, 320).astype(jnp.bfloat16))

    b1l = jnp.tile(c1_b[0, :10], 144).reshape(1, 1440).astype(jnp.bfloat16)
    b2l = jnp.tile(c2_b[0, :20], 16).reshape(1, 320).astype(jnp.bfloat16)
    f1 = fc1_w.astype(jnp.bfloat16)
    f2 = fc2_w.astype(jnp.bfloat16)

    const = lambda shape: pl.BlockSpec(shape, lambda i: tuple(0 for _ in shape))
    out = pl.pallas_call(
        _net_kernel,
        out_shape=jax.ShapeDtypeStruct((B, 128), jnp.float32),
        grid=(B // _BB,),
        in_specs=[pl.BlockSpec((_BB, 28, 28), lambda i: (i, 0, 0)),
                  const((4, 784, 1440)), const((4, 1440, 320)),
                  const((1, 1440)), const((1, 320)),
                  const((320, 128)), const((1, 128)),
                  const((128, 128)), const((1, 128))],
        out_specs=pl.BlockSpec((_BB, 128), lambda i: (i, 0)),
        compiler_params=pltpu.CompilerParams(dimension_semantics=("parallel",)),
    )(xr, g_all, h_all, b1l, b2l, f1, fc1_b, f2, fc2_b)
    return out[:, :10]


# speedup vs baseline: 1.3748x; 1.0355x over previous
import numpy as np
import ml_dtypes
import jax
import jax.numpy as jnp
from jax.experimental import pallas as pl
from jax.experimental.pallas import tpu as pltpu

_BB = 512


def _sel(n_out, n_in, k, off):
    r = np.zeros((n_out, n_in, k), np.float32)
    p = np.arange(n_out)[:, None]
    t = np.arange(k)[None, :]
    r[p, 2 * p + off + t, t] = 1.0
    return r


_R1 = [_sel(12, 28, 5, d) for d in (0, 1)]
_R2 = [_sel(4, 12, 5, d) for d in (0, 1)]

_R1S = np.stack(_R1)
_R2S = np.stack(_R2)


def _net_kernel(x_ref, g_ref, h_ref, b1_ref, b2_ref,
                f1_ref, fb1_ref, f2_ref, fb2_ref, o_ref):
    x = x_ref[...].astype(jnp.bfloat16).reshape(x_ref.shape[0], 784)

    z = jnp.dot(x, g_ref[0], preferred_element_type=jnp.float32).astype(jnp.bfloat16)
    z = jnp.maximum(z, jnp.dot(x, g_ref[1], preferred_element_type=jnp.float32).astype(jnp.bfloat16))
    z = jnp.maximum(z, jnp.dot(x, g_ref[2], preferred_element_type=jnp.float32).astype(jnp.bfloat16))
    z = jnp.maximum(z, jnp.dot(x, g_ref[3], preferred_element_type=jnp.float32).astype(jnp.bfloat16))
    a1 = jnp.maximum(z + b1_ref[...], 0.0)

    z = jnp.dot(a1, h_ref[0], preferred_element_type=jnp.float32).astype(jnp.bfloat16)
    z = jnp.maximum(z, jnp.dot(a1, h_ref[1], preferred_element_type=jnp.float32).astype(jnp.bfloat16))
    z = jnp.maximum(z, jnp.dot(a1, h_ref[2], preferred_element_type=jnp.float32).astype(jnp.bfloat16))
    z = jnp.maximum(z, jnp.dot(a1, h_ref[3], preferred_element_type=jnp.float32).astype(jnp.bfloat16))
    a2 = jnp.maximum(z + b2_ref[...], 0.0)

    h = jnp.dot(a2, f1_ref[...], preferred_element_type=jnp.float32) + fb1_ref[...]
    h = jnp.maximum(h, 0.0).astype(jnp.bfloat16)
    logits = jnp.dot(h, f2_ref[...], preferred_element_type=jnp.float32) + fb2_ref[...]
    m = jnp.max(logits, axis=-1, keepdims=True)
    lse = jnp.log(jnp.sum(jnp.exp(logits - m), axis=-1, keepdims=True)) + m
    o_ref[...] = logits - lse


def kernel(c1_w, c1_b, c2_w, c2_b, fc1_w, fc1_b, fc2_w, fc2_b, x):
    B = x.shape[0]
    xr = x.reshape(B, 28, 28)

    w1 = c1_w[:, :10].reshape(5, 5, 10)
    a1op = jnp.einsum('eqsw,hwc->eqshc', _R1S, w1)
    g_big = jnp.einsum('dprh,eqshc->dpreqsc', _R1S, a1op)
    g_all = (g_big.transpose(0, 3, 2, 5, 1, 4, 6)
             .reshape(4, 784, 1440).astype(jnp.bfloat16))
    w2 = c2_w[:, :20].reshape(5, 5, 10, 20)
    a2op = jnp.einsum('ebqw,hwic->ebqhic', _R2S, w2)
    h_big = jnp.einsum('daph,ebqhic->dapebqic', _R2S, a2op)
    h_all = (h_big.transpose(0, 3, 2, 5, 6, 1, 4, 7)
             .reshape(4, 1440, 320).astype(jnp.bfloat16))

    b1l = jnp.tile(c1_b[0, :10], 144).reshape(1, 1440).astype(jnp.bfloat16)
    b2l = jnp.tile(c2_b[0, :20], 16).reshape(1, 320).astype(jnp.bfloat16)
    f1 = fc1_w.astype(jnp.bfloat16)
    f2 = fc2_w.astype(jnp.bfloat16)

    const = lambda shape: pl.BlockSpec(shape, lambda i: tuple(0 for _ in shape))
    out = pl.pallas_call(
        _net_kernel,
        out_shape=jax.ShapeDtypeStruct((B, 128), jnp.float32),
        grid=(B // _BB,),
        in_specs=[pl.BlockSpec((_BB, 28, 28), lambda i: (i, 0, 0)),
                  const((4, 784, 1440)), const((4, 1440, 320)),
                  const((1, 1440)), const((1, 320)),
                  const((320, 128)), const((1, 128)),
                  const((128, 128)), const((1, 128))],
        out_specs=pl.BlockSpec((_BB, 128), lambda i: (i, 0)),
        compiler_params=pltpu.CompilerParams(dimension_semantics=("parallel",)),
    )(xr, g_all, h_all, b1l, b2l, f1, fc1_b, f2, fc2_b)
    return out[:, :10]


# speedup vs baseline: 1.4146x; 1.0290x over previous
import numpy as np
import ml_dtypes
import jax
import jax.numpy as jnp
from jax.experimental import pallas as pl
from jax.experimental.pallas import tpu as pltpu

_BB = 512


def _sel(n_out, n_in, k, off):
    r = np.zeros((n_out, n_in, k), np.float32)
    p = np.arange(n_out)[:, None]
    t = np.arange(k)[None, :]
    r[p, 2 * p + off + t, t] = 1.0
    return r


_R1 = [_sel(12, 28, 5, d) for d in (0, 1)]
_R2 = [_sel(4, 12, 5, d) for d in (0, 1)]

_R1S = np.stack(_R1)
_R2S = np.stack(_R2)
_R1SB = _R1S.astype(ml_dtypes.bfloat16)
_R2SB = _R2S.astype(ml_dtypes.bfloat16)


def _net_kernel(x_ref, g_ref, h_ref, b1_ref, b2_ref,
                f1_ref, fb1_ref, f2_ref, fb2_ref, o_ref):
    x = x_ref[...].astype(jnp.bfloat16).reshape(x_ref.shape[0], 784)

    z = jnp.dot(x, g_ref[0], preferred_element_type=jnp.float32).astype(jnp.bfloat16)
    z = jnp.maximum(z, jnp.dot(x, g_ref[1], preferred_element_type=jnp.float32).astype(jnp.bfloat16))
    z = jnp.maximum(z, jnp.dot(x, g_ref[2], preferred_element_type=jnp.float32).astype(jnp.bfloat16))
    z = jnp.maximum(z, jnp.dot(x, g_ref[3], preferred_element_type=jnp.float32).astype(jnp.bfloat16))
    a1 = jnp.maximum(z + b1_ref[...], 0.0)

    z = jnp.dot(a1, h_ref[0], preferred_element_type=jnp.float32).astype(jnp.bfloat16)
    z = jnp.maximum(z, jnp.dot(a1, h_ref[1], preferred_element_type=jnp.float32).astype(jnp.bfloat16))
    z = jnp.maximum(z, jnp.dot(a1, h_ref[2], preferred_element_type=jnp.float32).astype(jnp.bfloat16))
    z = jnp.maximum(z, jnp.dot(a1, h_ref[3], preferred_element_type=jnp.float32).astype(jnp.bfloat16))
    a2 = jnp.maximum(z + b2_ref[...], 0.0)

    h = jnp.dot(a2, f1_ref[...], preferred_element_type=jnp.float32) + fb1_ref[...]
    h = jnp.maximum(h, 0.0).astype(jnp.bfloat16)
    logits = jnp.dot(h, f2_ref[...], preferred_element_type=jnp.float32) + fb2_ref[...]
    m = jnp.max(logits, axis=-1, keepdims=True)
    lse = jnp.log(jnp.sum(jnp.exp(logits - m), axis=-1, keepdims=True)) + m
    o_ref[...] = logits - lse


def kernel(c1_w, c1_b, c2_w, c2_b, fc1_w, fc1_b, fc2_w, fc2_b, x):
    B = x.shape[0]
    xr = x.reshape(B, 28, 28)

    w1 = c1_w[:, :10].reshape(5, 5, 10)
    a1op = (jnp.einsum('eqsw,hwc->esqhc', _R1S, w1)
            .astype(jnp.bfloat16))
    g_big = jnp.einsum('dprh,esqhc->dpresqc', _R1SB, a1op)
    g_all = (g_big.transpose(0, 3, 2, 4, 1, 5, 6)
             .reshape(4, 784, 1440))
    w2 = c2_w[:, :20].reshape(5, 5, 10, 20)
    a2op = (jnp.einsum('ebqw,hwic->eqihbc', _R2S, w2)
            .astype(jnp.bfloat16))
    h_big = jnp.einsum('daph,eqihbc->dapeqibc', _R2SB, a2op)
    h_all = (h_big.transpose(0, 3, 2, 4, 5, 1, 6, 7)
             .reshape(4, 1440, 320))

    b1l = jnp.tile(c1_b[0, :10], 144).reshape(1, 1440).astype(jnp.bfloat16)
    b2l = jnp.tile(c2_b[0, :20], 16).reshape(1, 320).astype(jnp.bfloat16)
    f1 = fc1_w.astype(jnp.bfloat16)
    f2 = fc2_w.astype(jnp.bfloat16)

    const = lambda shape: pl.BlockSpec(shape, lambda i: tuple(0 for _ in shape))
    out = pl.pallas_call(
        _net_kernel,
        out_shape=jax.ShapeDtypeStruct((B, 128), jnp.float32),
        grid=(B // _BB,),
        in_specs=[pl.BlockSpec((_BB, 28, 28), lambda i: (i, 0, 0)),
                  const((4, 784, 1440)), const((4, 1440, 320)),
                  const((1, 1440)), const((1, 320)),
                  const((320, 128)), const((1, 128)),
                  const((128, 128)), const((1, 128))],
        out_specs=pl.BlockSpec((_BB, 128), lambda i: (i, 0)),
        compiler_params=pltpu.CompilerParams(dimension_semantics=("parallel",)),
    )(xr, g_all, h_all, b1l, b2l, f1, fc1_b, f2, fc2_b)
    return out[:, :10]


# speedup vs baseline: 2.3969x; 1.6944x over previous
import numpy as np
import ml_dtypes
import jax
import jax.numpy as jnp
from jax.experimental import pallas as pl
from jax.experimental.pallas import tpu as pltpu

_BB = 512


def _sel(n_out, n_in, k, off):
    r = np.zeros((n_out, n_in, k), np.float32)
    p = np.arange(n_out)[:, None]
    t = np.arange(k)[None, :]
    r[p, 2 * p + off + t, t] = 1.0
    return r


_R1S = np.stack([_sel(12, 28, 5, d) for d in (0, 1)])
_R2S = np.stack([_sel(4, 12, 5, d) for d in (0, 1)])


def _pack_kernel(a1op_ref, a2op_ref, g_ref, h_ref):
    g_ref[...] = jnp.zeros(g_ref.shape, g_ref.dtype)
    h_ref[...] = jnp.zeros(h_ref.shape, h_ref.dtype)
    for d in range(2):
        for p in range(12):
            for kh in range(5):
                r = 2 * p + d + kh
                g_ref[d, r * 28:(r + 1) * 28, p * 128:(p + 1) * 128] = a1op_ref[0, kh]
                pa = p - d - kh
                if pa >= 0 and pa < 8 and pa % 2 == 0:
                    h_ref[d, p * 128:p * 128 + 120,
                          (pa // 2) * 128:(pa // 2 + 1) * 128] = a2op_ref[0, kh]


def _net_kernel(x_ref, g_ref, h_ref, b1_ref, b2_ref,
                f1_ref, fb1_ref, f2_ref, fb2_ref, o_ref):
    x = x_ref[...].astype(jnp.bfloat16).reshape(x_ref.shape[0], 784)

    z = jnp.dot(x, g_ref[0], preferred_element_type=jnp.float32).astype(jnp.bfloat16)
    z = jnp.maximum(z, jnp.dot(x, g_ref[1], preferred_element_type=jnp.float32).astype(jnp.bfloat16))
    z = jnp.maximum(z, jnp.dot(x, g_ref[2], preferred_element_type=jnp.float32).astype(jnp.bfloat16))
    z = jnp.maximum(z, jnp.dot(x, g_ref[3], preferred_element_type=jnp.float32).astype(jnp.bfloat16))
    a1 = jnp.maximum(z + b1_ref[...], 0.0)

    z = jnp.dot(a1, h_ref[0], preferred_element_type=jnp.float32).astype(jnp.bfloat16)
    z = jnp.maximum(z, jnp.dot(a1, h_ref[1], preferred_element_type=jnp.float32).astype(jnp.bfloat16))
    z = jnp.maximum(z, jnp.dot(a1, h_ref[2], preferred_element_type=jnp.float32).astype(jnp.bfloat16))
    z = jnp.maximum(z, jnp.dot(a1, h_ref[3], preferred_element_type=jnp.float32).astype(jnp.bfloat16))
    a2 = jnp.maximum(z + b2_ref[...], 0.0)

    h = jnp.dot(a2, f1_ref[...], preferred_element_type=jnp.float32) + fb1_ref[...]
    h = jnp.maximum(h, 0.0).astype(jnp.bfloat16)
    logits = jnp.dot(h, f2_ref[...], preferred_element_type=jnp.float32) + fb2_ref[...]
    m = jnp.max(logits, axis=-1, keepdims=True)
    lse = jnp.log(jnp.sum(jnp.exp(logits - m), axis=-1, keepdims=True)) + m
    o_ref[...] = logits - lse


def kernel(c1_w, c1_b, c2_w, c2_b, fc1_w, fc1_b, fc2_w, fc2_b, x):
    B = x.shape[0]
    xr = x.reshape(B, 28, 28)

    w1 = c1_w[:, :10].reshape(5, 5, 10)
    a1op = (jnp.einsum('eqsw,hwc->ehsqc', _R1S, w1)
            .reshape(2, 5, 28, 120).astype(jnp.bfloat16))
    a1op = jnp.pad(a1op, ((0, 0), (0, 0), (0, 0), (0, 8)))
    w2 = c2_w[:, :20].reshape(5, 5, 10, 20)
    a2op = (jnp.einsum('ebqw,hwic->ehqibc', _R2S, w2)
            .reshape(2, 5, 120, 80).astype(jnp.bfloat16))
    a2op = jnp.pad(a2op, ((0, 0), (0, 0), (0, 0), (0, 48)))

    g_all, h_all = pl.pallas_call(
        _pack_kernel,
        out_shape=(jax.ShapeDtypeStruct((4, 784, 1536), jnp.bfloat16),
                   jax.ShapeDtypeStruct((4, 1536, 512), jnp.bfloat16)),
        grid=(2,),
        in_specs=[pl.BlockSpec((1, 5, 28, 128), lambda e: (e, 0, 0, 0)),
                  pl.BlockSpec((1, 5, 120, 128), lambda e: (e, 0, 0, 0))],
        out_specs=(pl.BlockSpec((2, 784, 1536), lambda e: (e, 0, 0)),
                   pl.BlockSpec((2, 1536, 512), lambda e: (e, 0, 0))),
        compiler_params=pltpu.CompilerParams(dimension_semantics=("parallel",)),
    )(a1op, a2op)

    b1l = jnp.pad(jnp.tile(c1_b[0, :10], 12).reshape(1, 120), ((0, 0), (0, 8)))
    b1l = jnp.tile(b1l, (1, 12)).astype(jnp.bfloat16)
    b2l = jnp.pad(jnp.tile(c2_b[0, :20], 4).reshape(1, 80), ((0, 0), (0, 48)))
    b2l = jnp.tile(b2l, (1, 4)).astype(jnp.bfloat16)
    f1 = jnp.pad(fc1_w.reshape(4, 80, 128), ((0, 0), (0, 48), (0, 0)))
    f1 = f1.reshape(512, 128).astype(jnp.bfloat16)
    f2 = fc2_w.astype(jnp.bfloat16)

    const = lambda shape: pl.BlockSpec(shape, lambda i: tuple(0 for _ in shape))
    out = pl.pallas_call(
        _net_kernel,
        out_shape=jax.ShapeDtypeStruct((B, 128), jnp.float32),
        grid=(B // _BB,),
        in_specs=[pl.BlockSpec((_BB, 28, 28), lambda i: (i, 0, 0)),
                  const((4, 784, 1536)), const((4, 1536, 512)),
                  const((1, 1536)), const((1, 512)),
                  const((512, 128)), const((1, 128)),
                  const((128, 128)), const((1, 128))],
        out_specs=pl.BlockSpec((_BB, 128), lambda i: (i, 0)),
        compiler_params=pltpu.CompilerParams(dimension_semantics=("parallel",)),
    )(xr, g_all, h_all, b1l, b2l, f1, fc1_b, f2, fc2_b)
    return out[:, :10]
